# baseline (speedup 1.0000x reference)
import functools

import jax
import jax.numpy as jnp
from jax import lax
from jax.experimental import pallas as pl
from jax.experimental.pallas import tpu as pltpu
from jax.experimental.pallas import tpu_sc as plsc

D = 4
W = 4194304
B = 16384
H = 64
M31 = 2147483647
_PRIMES = (1000003, 1000033, 1000037, 1000039)
_OFFS = (12582917, 25165843, 50331653, 100663319)

NC, NS, L = 2, 16, 16
NW = NC * NS
BW = B // NW
NV = BW // L
CHUNK = 128
NCH = (D * BW) // CHUNK

@functools.cache
def _sc_hash_gather_fn(paired: bool):
    mesh = plsc.VectorSubcoreMesh(
        core_axis_name="c", subcore_axis_name="s",
        num_cores=NC, num_subcores=NS)
    xw = 2 if paired else 1
    return functools.partial(
        pl.kernel,
        out_type=jax.ShapeDtypeStruct((D, B), jnp.float32),
        mesh=mesh,
        scratch_types=[
            pltpu.VMEM((BW * xw,), jnp.int32),
            pltpu.VMEM((BW,), jnp.int32),
            pltpu.VMEM((NCH, CHUNK), jnp.int32),
            pltpu.VMEM((D * BW,), jnp.float32),
            pltpu.SemaphoreType.DMA,
        ],
        compiler_params=pltpu.CompilerParams(needs_layout_passes=False),
    )(functools.partial(_sc_hash_gather, paired))


def _sc_hash_gather(paired, x_hbm, sk_hbm, outT_hbm,
                    xr_v, x_v, idx_v, vals_v, sem):
    wid = lax.axis_index("s") * NC + lax.axis_index("c")
    base = wid * BW
    if paired:
        pltpu.sync_copy(x_hbm.at[pl.ds(base * 2, BW * 2)], xr_v)
        even = lax.iota(jnp.int32, L) * 2
        for j in range(NV):
            x_v[pl.ds(j * L, L)] = plsc.load_gather(xr_v, [even + (j * 2 * L)])
    else:
        pltpu.sync_copy(x_hbm.at[pl.ds(base, BW)], x_v)
    jpc = CHUNK // L
    cpd = NCH // D

    for d in range(D):

        def chunk_body(cj, carry, d=d):
            def vec_body(jj, carry2, cj=cj):
                j = cj * jpc + jj
                xv = x_v[pl.ds(j * L, L)]
                xh = xv >> 10
                xl = xv & 1023
                a = xh * _PRIMES[d]
                cc = xl * _PRIMES[d] + _OFFS[d]
                t = (a >> 21) + ((a & 0x1FFFFF) << 10)
                s = t + cc
                r = jnp.where((s < 0) | (s >= M31), s - M31, s)
                h = (((r & (W - 128)) << 2) | (d << 7)) | (r & 127)
                idx_v[d * cpd + cj, pl.ds(jj * L, L)] = h
                return carry2

            lax.fori_loop(jnp.int32(0), jnp.int32(jpc), vec_body,
                          jnp.int32(0))
            c = d * cpd + cj
            pltpu.async_copy(sk_hbm.at[idx_v.at[c]],
                             vals_v.at[pl.ds(c * CHUNK, CHUNK)], sem)
            return carry

        lax.fori_loop(jnp.int32(0), jnp.int32(cpd), chunk_body,
                      jnp.int32(0))

    def drain_body(c, carry):
        pltpu.make_async_copy(sk_hbm.at[idx_v.at[c]],
                              vals_v.at[pl.ds(c * CHUNK, CHUNK)],
                              sem).wait()
        return carry

    lax.fori_loop(jnp.int32(0), jnp.int32(NCH), drain_body, jnp.int32(0))
    for d in range(D):
        pltpu.sync_copy(vals_v.at[pl.ds(d * BW, BW)],
                        outT_hbm.at[jnp.int32(d), pl.ds(base, BW)])


def _tc_decode(valsT, fs, w1t, b1c, w2t, b2c, pred_o, rf_o, flg_o, is_o):
    ri = valsT[...]
    fs2 = fs[...]
    rf = jnp.min(ri, axis=0, keepdims=True)
    mean = jnp.mean(rf)
    flg = (rf > mean).astype(jnp.float32)
    denom = fs2 + 1e-6
    featT = jnp.concatenate([ri / denom, rf / denom, flg], axis=0)
    hT = jnp.dot(w1t[...], featT, preferred_element_type=jnp.float32)
    hT = jnp.maximum(hT + b1c[...], 0.0)
    outT = jnp.dot(w2t[...], hT, preferred_element_type=jnp.float32)
    outT = outT + b2c[...]
    pred_o[...] = outT[0:1, :] * fs2
    rf_o[...] = rf
    flg_o[...] = flg
    is_o[...] = 1.0 / (1.0 + jnp.exp(-outT[1:2, :]))


def kernel(batch_query_x, batch_freqs_sum, sketch, W1, b1, W2, b2):
    xin = batch_query_x.astype(jnp.int32)
    skf = (sketch.astype(jnp.float32)
           .reshape(D, W // 128, 128)
           .transpose(1, 0, 2)
           .reshape(D * W))
    valsT = _sc_hash_gather_fn(False)(xin, skf)
    read_info = valsT.T
    fs2 = batch_freqs_sum.astype(jnp.float32).reshape(1, B)
    w1t = W1.astype(jnp.float32).T
    b1c = b1.astype(jnp.float32).reshape(H, 1)
    w2t = W2.astype(jnp.float32).T
    b2c = b2.astype(jnp.float32).reshape(2, 1)
    pred, rf, flg, isv = pl.pallas_call(
        _tc_decode,
        out_shape=[jax.ShapeDtypeStruct((1, B), jnp.float32)] * 4,
    )(valsT, fs2, w1t, b1c, w2t, b2c)
    return (pred.reshape(B), rf.reshape(B), flg.reshape(B) > 0.0,
            read_info, isv.reshape(B))


# speedup vs baseline: 1.0140x; 1.0140x over previous
import functools

import jax
import jax.numpy as jnp
from jax import lax
from jax.experimental import pallas as pl
from jax.experimental.pallas import tpu as pltpu
from jax.experimental.pallas import tpu_sc as plsc

D = 4
W = 4194304
B = 16384
H = 64
M31 = 2147483647
_PRIMES = (1000003, 1000033, 1000037, 1000039)
_OFFS = (12582917, 25165843, 50331653, 100663319)

NC, NS, L = 2, 16, 16
NW = NC * NS
BW = B // NW
NV = BW // L
CHUNK = 128
NCH = (D * BW) // CHUNK

@functools.cache
def _sc_hash_gather_fn(paired: bool):
    mesh = plsc.VectorSubcoreMesh(
        core_axis_name="c", subcore_axis_name="s",
        num_cores=NC, num_subcores=NS)
    xw = 2 if paired else 1
    return functools.partial(
        pl.kernel,
        out_type=jax.ShapeDtypeStruct((D, B), jnp.float32),
        mesh=mesh,
        scratch_types=[
            pltpu.VMEM((BW * xw,), jnp.int32),
            pltpu.VMEM((BW,), jnp.int32),
            pltpu.VMEM((NCH, CHUNK), jnp.int32),
            pltpu.VMEM((D * BW,), jnp.float32),
            pltpu.SemaphoreType.DMA,
        ],
        compiler_params=pltpu.CompilerParams(needs_layout_passes=False),
    )(functools.partial(_sc_hash_gather, paired))


def _sc_hash_gather(paired, x_hbm, sk_hbm, outT_hbm,
                    xr_v, x_v, idx_v, vals_v, sem):
    wid = lax.axis_index("s") * NC + lax.axis_index("c")
    base = wid * BW
    if paired:
        pltpu.sync_copy(x_hbm.at[pl.ds(base * 2, BW * 2)], xr_v)
        even = lax.iota(jnp.int32, L) * 2
        for j in range(NV):
            x_v[pl.ds(j * L, L)] = plsc.load_gather(xr_v, [even + (j * 2 * L)])
    else:
        pltpu.sync_copy(x_hbm.at[pl.ds(base, BW)], x_v)
    jpc = CHUNK // L

    def hash_body(j, carry):
        xv = x_v[pl.ds(j * L, L)]
        xh = xv >> 10
        xl = xv & 1023
        row0 = j // jpc
        col = (j % jpc) * L
        for d in range(D):
            a = xh * _PRIMES[d]
            cc = xl * _PRIMES[d] + _OFFS[d]
            t = (a >> 21) + ((a & 0x1FFFFF) << 10)
            s = t + cc
            r = jnp.where((s < 0) | (s >= M31), s - M31, s)
            h = (((r & (W - 128)) << 2) | (d << 7)) | (r & 127)
            idx_v[row0 + d * (NCH // D), pl.ds(col, L)] = h
        return carry

    lax.fori_loop(jnp.int32(0), jnp.int32(NV), hash_body, jnp.int32(0))
    copies = [
        pltpu.async_copy(sk_hbm.at[idx_v.at[jnp.int32(c)]],
                         vals_v.at[pl.ds(c * CHUNK, CHUNK)], sem)
        for c in range(NCH)
    ]
    for cp in copies:
        cp.wait()
    for d in range(D):
        pltpu.sync_copy(vals_v.at[pl.ds(d * BW, BW)],
                        outT_hbm.at[jnp.int32(d), pl.ds(base, BW)])


def _tc_decode(valsT, fs, w1t, b1c, w2t, b2c, pred_o, rf_o, flg_o, is_o):
    ri = valsT[...]
    fs2 = fs[...]
    rf = jnp.min(ri, axis=0, keepdims=True)
    mean = jnp.mean(rf)
    flg_b = rf > mean
    flg = flg_b.astype(jnp.float32)
    denom = fs2 + 1e-6
    featT = jnp.concatenate([ri / denom, rf / denom, flg], axis=0)
    hT = jnp.dot(w1t[...], featT, preferred_element_type=jnp.float32)
    hT = jnp.maximum(hT + b1c[...], 0.0)
    outT = jnp.dot(w2t[...], hT, preferred_element_type=jnp.float32)
    outT = outT + b2c[...]
    pred_o[...] = outT[0:1, :] * fs2
    rf_o[...] = rf
    flg_o[...] = flg_b
    is_o[...] = 1.0 / (1.0 + jnp.exp(-outT[1:2, :]))


def kernel(batch_query_x, batch_freqs_sum, sketch, W1, b1, W2, b2):
    xin = batch_query_x.astype(jnp.int32)
    skf = (sketch.astype(jnp.float32)
           .reshape(D, W // 128, 128)
           .transpose(1, 0, 2)
           .reshape(D * W))
    valsT = _sc_hash_gather_fn(False)(xin, skf)
    read_info = valsT.T
    fs2 = batch_freqs_sum.astype(jnp.float32).reshape(1, B)
    w1t = W1.astype(jnp.float32).T
    b1c = b1.astype(jnp.float32).reshape(H, 1)
    w2t = W2.astype(jnp.float32).T
    b2c = b2.astype(jnp.float32).reshape(2, 1)
    pred, rf, flg, isv = pl.pallas_call(
        _tc_decode,
        out_shape=[jax.ShapeDtypeStruct((1, B), jnp.float32),
                   jax.ShapeDtypeStruct((1, B), jnp.float32),
                   jax.ShapeDtypeStruct((1, B), jnp.bool_),
                   jax.ShapeDtypeStruct((1, B), jnp.float32)],
    )(valsT, fs2, w1t, b1c, w2t, b2c)
    return (pred.reshape(B), rf.reshape(B), flg.reshape(B),
            read_info, isv.reshape(B))
